# baseline (speedup 1.0000x reference)
import dataclasses
import functools

import jax
import jax.numpy as jnp
from jax import lax
from jax.experimental import pallas as pl
from jax.experimental.pallas import tpu as pltpu
from jax.experimental.pallas import tpu_sc as plsc

_N = 10000
_E = 320000
_D = 128
_G = 64

_NC = 2
_NS = 16
_NW = _NC * _NS
_EPW = _E // _NW
_CH = 100
_GC = 10
_NG = _EPW // (_GC * _CH)
_RSTRIDE = 624
_RWIN = 640
_ZR = 16
_L = 16

_F32 = jnp.float32


def _make_sc_msgpass(with_cnt):
  mesh = plsc.VectorSubcoreMesh(core_axis_name="c", subcore_axis_name="s")
  out_type = [jax.ShapeDtypeStruct((_NC, _N, _D), _F32)]
  nbuf = 2 if with_cnt else 3
  scratch = [
      pltpu.VMEM_SHARED((_N, _D), _F32),
      pltpu.VMEM((_ZR, _D), _F32),
      pltpu.VMEM((_GC, _CH), jnp.int32),
      pltpu.VMEM((_GC, _CH), jnp.int32),
  ]
  for _ in range(nbuf):
    scratch += [
        pltpu.VMEM((_CH, _D), _F32),
        pltpu.SemaphoreType.DMA,
        pltpu.SemaphoreType.DMA,
    ]
  if with_cnt:
    out_type.append(jax.ShapeDtypeStruct((_NC, _NS, _N), _F32))
    scratch.append(pltpu.VMEM((_N,), _F32))
  cp = pltpu.CompilerParams()
  if "needs_layout_passes" in pltpu.CompilerParams.__dataclass_fields__:
    cp = dataclasses.replace(cp, needs_layout_passes=False)

  def body(x_hbm, src_hbm, dst_hbm, *rest):
    if with_cnt:
      (agg_hbm, cnt_hbm, agg_sh, zbuf, src_v, dst_v, *ring, hist) = rest
    else:
      (agg_hbm, agg_sh, zbuf, src_v, dst_v, *ring) = rest
    bufs = tuple(tuple(ring[3 * b:3 * b + 3]) for b in range(nbuf))
    c = lax.axis_index("c")
    s = lax.axis_index("s")

    @pl.loop(0, _ZR)
    def _(i):
      @pl.loop(0, _D // _L)
      def _(j):
        zbuf.at[i, pl.ds(j * _L, _L)][...] = jnp.zeros((_L,), _F32)

    if with_cnt:
      @pl.loop(0, _N // _L)
      def _(i):
        hist.at[pl.ds(i * _L, _L)][...] = jnp.zeros((_L,), _F32)

    rbase = s * _RSTRIDE

    @pl.loop(0, _RWIN // _ZR)
    def _(k):
      pltpu.sync_copy(zbuf, agg_sh.at[pl.ds(rbase + k * _ZR, _ZR)])

    plsc.subcore_barrier()

    wid = c * _NS + s

    def g_start(i, b):
      rows, gsem, _ = bufs[b]
      pltpu.async_copy(x_hbm.at[src_v.at[i]], rows, gsem)

    def g_wait(i, b):
      rows, gsem, _ = bufs[b]
      pltpu.make_async_copy(x_hbm.at[src_v.at[i]], rows, gsem).wait()

    def s_start(i, b):
      rows, _, ssem = bufs[b]
      pltpu.async_copy(rows, agg_sh.at[dst_v.at[i]], ssem,
                       add=True)
      if with_cnt:
        @pl.loop(0, _CH // _L)
        def _(k):
          ii = dst_v.at[i, pl.ds(k * _L, _L)][...]
          plsc.addupdate_scatter(hist, [ii], jnp.ones((_L,), _F32))

    def s_wait(i, b):
      rows, _, ssem = bufs[b]
      pltpu.make_async_copy(rows, agg_sh.at[dst_v.at[i]], ssem).wait()

    def s_sync(i, b):
      s_start(i, b)
      s_wait(i, b)

    @pl.loop(0, _NG)
    def _(g):
      pltpu.sync_copy(src_hbm.at[wid, g], src_v)
      pltpu.sync_copy(dst_hbm.at[wid, g], dst_v)
      if with_cnt:
        g_start(0, 0)

        @pl.loop(0, (_GC - 2) // 2)
        def _(jj):
          i0 = 2 * jj
          g_start(i0 + 1, 1)
          g_wait(i0, 0)
          s_sync(i0, 0)
          g_start(i0 + 2, 0)
          g_wait(i0 + 1, 1)
          s_sync(i0 + 1, 1)

        g_start(_GC - 1, 1)
        g_wait(_GC - 2, 0)
        s_sync(_GC - 2, 0)
        g_wait(_GC - 1, 1)
        s_sync(_GC - 1, 1)
      else:
        g_start(0, 0)
        g_start(1, 1)
        g_wait(0, 0)
        s_start(0, 0)
        g_start(2, 2)
        g_wait(1, 1)
        s_start(1, 1)

        @pl.loop(0, (_GC - 4) // 3)
        def _(jj):
          c0 = 3 * jj
          s_wait(c0, 0)
          g_start(c0 + 3, 0)
          g_wait(c0 + 2, 2)
          s_start(c0 + 2, 2)
          s_wait(c0 + 1, 1)
          g_start(c0 + 4, 1)
          g_wait(c0 + 3, 0)
          s_start(c0 + 3, 0)
          s_wait(c0 + 2, 2)
          g_start(c0 + 5, 2)
          g_wait(c0 + 4, 1)
          s_start(c0 + 4, 1)

        s_wait(_GC - 4, 0)
        g_start(_GC - 1, 0)
        g_wait(_GC - 2, 2)
        s_start(_GC - 2, 2)
        g_wait(_GC - 1, 0)
        s_start(_GC - 1, 0)
        s_wait(_GC - 3, 1)
        s_wait(_GC - 2, 2)
        s_wait(_GC - 1, 0)

    plsc.subcore_barrier()

    pltpu.sync_copy(agg_sh.at[pl.ds(rbase, _RWIN)],
                    agg_hbm.at[c, pl.ds(rbase, _RWIN)])
    if with_cnt:
      pltpu.sync_copy(hist, cnt_hbm.at[c, s])

  kern = pl.kernel(body, out_type=tuple(out_type), mesh=mesh,
                   scratch_types=scratch, compiler_params=cp)

  def call(x, src4, dst4):
    return kern(x, src4, dst4)

  return call


_sc_layer1 = _make_sc_msgpass(True)
_sc_layer2 = _make_sc_msgpass(False)

_HI = lax.Precision.HIGHEST
_R = 2000
_NB = _N // _R


def _cnt_col(cnt_ref, i):
  del i
  cnt = cnt_ref[:, 0, 0, :]
  return lax.dot_general(cnt, jnp.ones((_NW, 1), _F32),
                         (((0,), (0,)), ((), ())),
                         precision=_HI, preferred_element_type=_F32)


def _root_body(x_ref, w_ref, b_ref, out_ref):
  out_ref[...] = (jnp.dot(x_ref[...], w_ref[...], precision=_HI,
                          preferred_element_type=_F32) + b_ref[...])


def _tc1_body(agg_ref, cnt_ref, xr_ref, w1l_ref, h_ref):
  cnt = _cnt_col(cnt_ref, pl.program_id(0))
  mean = (agg_ref[0] + agg_ref[1]) / jnp.maximum(cnt, 1.0)
  h = (jnp.dot(mean, w1l_ref[...], precision=_HI, preferred_element_type=_F32)
       + xr_ref[...])
  h_ref[...] = jnp.maximum(h, 0.0)


def _tc2_body(agg_ref, cnt_ref, hr_ref, w2l_ref, batch_ref,
              pooled_ref, h2_ref, cg_ref):
  i = pl.program_id(0)
  cnt = _cnt_col(cnt_ref, i)
  mean = (agg_ref[0] + agg_ref[1]) / jnp.maximum(cnt, 1.0)
  h2 = (jnp.dot(mean, w2l_ref[...], precision=_HI, preferred_element_type=_F32)
        + hr_ref[...])
  h2_ref[...] = h2
  sel = (lax.broadcasted_iota(jnp.int32, (_G, _R), 0)
         == batch_ref[0]).astype(_F32)
  psum = jnp.dot(sel, h2, precision=_HI, preferred_element_type=_F32)
  cg = jnp.sum(sel, axis=1, keepdims=True)

  @pl.when(i == 0)
  def _():
    pooled_ref[...] = jnp.zeros_like(pooled_ref)
    cg_ref[...] = jnp.zeros_like(cg_ref)

  pooled_ref[...] += psum
  cg_ref[...] += cg

  @pl.when(i == _NB - 1)
  def _():
    pooled_ref[...] = pooled_ref[...] / jnp.maximum(cg_ref[...], 1.0)


_w_spec = pl.BlockSpec((_D, _D), lambda i: (0, 0))
_b_spec = pl.BlockSpec((1, _D), lambda i: (0, 0))
_agg_spec = pl.BlockSpec((_NC, _R, _D), lambda i: (0, i, 0))
_cnt_spec = pl.BlockSpec((_NW, 1, 1, _R), lambda i: (0, i, 0, 0))
_row_spec = pl.BlockSpec((_R, _D), lambda i: (i, 0))


def _root_mm(x, W, b):
  return pl.pallas_call(
      _root_body,
      grid=(_NB,),
      in_specs=[_row_spec, _w_spec, _b_spec],
      out_specs=_row_spec,
      out_shape=jax.ShapeDtypeStruct((_N, _D), _F32),
  )(x, W, b.reshape(1, _D))


def kernel(x, edge_index, batch, W1l, b1l, W1r, W2l, b2l, W2r):
  src4 = edge_index[0].reshape(_NW, _NG, _GC, _CH)
  dst4 = edge_index[1].reshape(_NW, _NG, _GC, _CH)
  agg1, cnt = _sc_layer1(x, src4, dst4)
  xr = _root_mm(x, W1r, b1l)
  h = pl.pallas_call(
      _tc1_body,
      grid=(_NB,),
      in_specs=[_agg_spec, _cnt_spec, _row_spec, _w_spec],
      out_specs=_row_spec,
      out_shape=jax.ShapeDtypeStruct((_N, _D), _F32),
  )(agg1, cnt.reshape(_NW, _NB, 1, _R), xr, W1l)
  (agg2,) = _sc_layer2(h, src4, dst4)
  hr = _root_mm(h, W2r, b2l)
  pooled, h2 = pl.pallas_call(
      _tc2_body,
      grid=(_NB,),
      in_specs=[_agg_spec, _cnt_spec, _row_spec, _w_spec,
                pl.BlockSpec((1, 1, _R), lambda i: (i, 0, 0))],
      out_specs=(pl.BlockSpec((_G, _D), lambda i: (0, 0)), _row_spec),
      out_shape=(jax.ShapeDtypeStruct((_G, _D), _F32),
                 jax.ShapeDtypeStruct((_N, _D), _F32)),
      scratch_shapes=[pltpu.VMEM((_G, 1), _F32)],
  )(agg2, cnt.reshape(_NW, _NB, 1, _R), hr, W2l, batch.reshape(_NB, 1, _R))
  return (pooled, h2)


# speedup vs baseline: 1.1180x; 1.1180x over previous
import dataclasses
import functools

import jax
import jax.numpy as jnp
from jax import lax
from jax.experimental import pallas as pl
from jax.experimental.pallas import tpu as pltpu
from jax.experimental.pallas import tpu_sc as plsc

_N = 10000
_E = 320000
_D = 128
_G = 64

_NC = 2
_NS = 16
_NW = _NC * _NS
_EPW = _E // _NW
_CH = 80
_M = _EPW // _CH
_RSTRIDE = 624
_RWIN = 640
_ZR = 16
_L = 16

_F32 = jnp.float32


def _make_sc_msgpass(with_cnt):
  mesh = plsc.VectorSubcoreMesh(core_axis_name="c", subcore_axis_name="s")
  out_type = [jax.ShapeDtypeStruct((_NC, _N, _D), _F32)]
  scratch = [
      pltpu.VMEM_SHARED((_N, _D), _F32),
      pltpu.VMEM((_ZR, _D), _F32),
  ]
  for _ in range(3):
    scratch += [
        pltpu.VMEM((_CH,), jnp.int32),
        pltpu.VMEM((_CH,), jnp.int32),
        pltpu.VMEM((_CH, _D), _F32),
        pltpu.SemaphoreType.DMA,
        pltpu.SemaphoreType.DMA,
        pltpu.SemaphoreType.DMA,
    ]
  if with_cnt:
    out_type.append(jax.ShapeDtypeStruct((_NC, _NS, _N), _F32))
    scratch.append(pltpu.VMEM((_N,), _F32))
  cp = pltpu.CompilerParams()
  if "needs_layout_passes" in pltpu.CompilerParams.__dataclass_fields__:
    cp = dataclasses.replace(cp, needs_layout_passes=False)

  def body(x_hbm, edge_hbm, *rest):
    if with_cnt:
      (agg_hbm, cnt_hbm, agg_sh, zbuf, *ring, hist) = rest
    else:
      (agg_hbm, agg_sh, zbuf, *ring) = rest
    bufs = (tuple(ring[0:6]), tuple(ring[6:12]), tuple(ring[12:18]))
    c = lax.axis_index("c")
    s = lax.axis_index("s")

    @pl.loop(0, _ZR)
    def _(i):
      @pl.loop(0, _D // _L)
      def _(j):
        zbuf.at[i, pl.ds(j * _L, _L)][...] = jnp.zeros((_L,), _F32)

    if with_cnt:
      @pl.loop(0, _N // _L)
      def _(i):
        hist.at[pl.ds(i * _L, _L)][...] = jnp.zeros((_L,), _F32)

    rbase = s * _RSTRIDE

    @pl.loop(0, _RWIN // _ZR)
    def _(k):
      pltpu.sync_copy(zbuf, agg_sh.at[pl.ds(rbase + k * _ZR, _ZR)])

    plsc.subcore_barrier()

    wid = c * _NS + s
    ebase = wid * _EPW

    def i_start(ch, b):
      src_v, dst_v, _, isem, _, _ = bufs[b]
      off = ebase + ch * _CH
      pltpu.async_copy(edge_hbm.at[pl.ds(off, _CH)], src_v, isem)
      pltpu.async_copy(edge_hbm.at[pl.ds(_E + off, _CH)], dst_v, isem)

    def i_wait(ch, b):
      src_v, dst_v, _, isem, _, _ = bufs[b]
      off = ebase + ch * _CH
      pltpu.make_async_copy(edge_hbm.at[pl.ds(off, _CH)], src_v,
                            isem).wait()
      pltpu.make_async_copy(edge_hbm.at[pl.ds(_E + off, _CH)], dst_v,
                            isem).wait()

    def g_start(b):
      src_v, _, rows, _, gsem, _ = bufs[b]
      pltpu.async_copy(x_hbm.at[src_v], rows, gsem)

    def g_wait(b):
      src_v, _, rows, _, gsem, _ = bufs[b]
      pltpu.make_async_copy(x_hbm.at[src_v], rows, gsem).wait()

    def s_start(b):
      _, dst_v, rows, _, _, ssem = bufs[b]
      pltpu.async_copy(rows, agg_sh.at[dst_v], ssem,
                       add=True)
      if with_cnt:
        @pl.loop(0, _CH // _L)
        def _(k):
          ii = dst_v.at[pl.ds(k * _L, _L)][...]
          plsc.addupdate_scatter(hist, [ii], jnp.ones((_L,), _F32))

    def s_wait(b):
      _, dst_v, rows, _, _, ssem = bufs[b]
      pltpu.make_async_copy(rows, agg_sh.at[dst_v], ssem).wait()

    i_start(0, 0)
    i_start(1, 1)
    i_wait(0, 0)
    g_start(0)
    i_start(2, 2)
    i_wait(1, 1)
    g_start(1)
    g_wait(0)
    s_start(0)

    @pl.loop(0, (_M - 5) // 3)
    def _(jj):
      c0 = 3 * jj
      s_wait(0)
      i_start(c0 + 3, 0)
      i_wait(c0 + 2, 2)
      g_start(2)
      g_wait(1)
      s_start(1)

      s_wait(1)
      i_start(c0 + 4, 1)
      i_wait(c0 + 3, 0)
      g_start(0)
      g_wait(2)
      s_start(2)

      s_wait(2)
      i_start(c0 + 5, 2)
      i_wait(c0 + 4, 1)
      g_start(1)
      g_wait(0)
      s_start(0)

    s_wait(0)
    i_start(_M - 2, 0)
    i_wait(_M - 3, 2)
    g_start(2)
    g_wait(1)
    s_start(1)

    s_wait(1)
    i_start(_M - 1, 1)
    i_wait(_M - 2, 0)
    g_start(0)
    g_wait(2)
    s_start(2)

    s_wait(2)
    i_wait(_M - 1, 1)
    g_start(1)
    g_wait(0)
    s_start(0)

    g_wait(1)
    s_start(1)

    s_wait(0)
    s_wait(1)

    plsc.subcore_barrier()

    pltpu.sync_copy(agg_sh.at[pl.ds(rbase, _RWIN)],
                    agg_hbm.at[c, pl.ds(rbase, _RWIN)])
    if with_cnt:
      pltpu.sync_copy(hist, cnt_hbm.at[c, s])

  kern = pl.kernel(body, out_type=tuple(out_type), mesh=mesh,
                   scratch_types=scratch, compiler_params=cp)

  return kern


_sc_layer1 = _make_sc_msgpass(True)
_sc_layer2 = _make_sc_msgpass(False)

_HI = lax.Precision.HIGHEST
_R = 2000
_NB = _N // _R


def _cnt_col(cnt_ref, i):
  del i
  cnt = cnt_ref[:, 0, 0, :]
  return lax.dot_general(cnt, jnp.ones((_NW, 1), _F32),
                         (((0,), (0,)), ((), ())),
                         precision=_HI, preferred_element_type=_F32)


def _root_body(x_ref, w_ref, b_ref, out_ref):
  out_ref[...] = (jnp.dot(x_ref[...], w_ref[...], precision=_HI,
                          preferred_element_type=_F32) + b_ref[...])


def _tc1_body(agg_ref, cnt_ref, xr_ref, w1l_ref, h_ref):
  cnt = _cnt_col(cnt_ref, pl.program_id(0))
  mean = (agg_ref[0] + agg_ref[1]) / jnp.maximum(cnt, 1.0)
  h = (jnp.dot(mean, w1l_ref[...], precision=_HI, preferred_element_type=_F32)
       + xr_ref[...])
  h_ref[...] = jnp.maximum(h, 0.0)


def _tc2_body(agg_ref, cnt_ref, hr_ref, w2l_ref, batch_ref,
              pooled_ref, h2_ref, cg_ref):
  i = pl.program_id(0)
  cnt = _cnt_col(cnt_ref, i)
  mean = (agg_ref[0] + agg_ref[1]) / jnp.maximum(cnt, 1.0)
  h2 = (jnp.dot(mean, w2l_ref[...], precision=_HI, preferred_element_type=_F32)
        + hr_ref[...])
  h2_ref[...] = h2
  sel = (lax.broadcasted_iota(jnp.int32, (_G, _R), 0)
         == batch_ref[0]).astype(_F32)
  psum = jnp.dot(sel, h2, precision=_HI, preferred_element_type=_F32)
  cg = jnp.sum(sel, axis=1, keepdims=True)

  @pl.when(i == 0)
  def _():
    pooled_ref[...] = jnp.zeros_like(pooled_ref)
    cg_ref[...] = jnp.zeros_like(cg_ref)

  pooled_ref[...] += psum
  cg_ref[...] += cg

  @pl.when(i == _NB - 1)
  def _():
    pooled_ref[...] = pooled_ref[...] / jnp.maximum(cg_ref[...], 1.0)


_w_spec = pl.BlockSpec((_D, _D), lambda i: (0, 0))
_b_spec = pl.BlockSpec((1, _D), lambda i: (0, 0))
_agg_spec = pl.BlockSpec((_NC, _R, _D), lambda i: (0, i, 0))
_cnt_spec = pl.BlockSpec((_NW, 1, 1, _R), lambda i: (0, i, 0, 0))
_row_spec = pl.BlockSpec((_R, _D), lambda i: (i, 0))


def _root_mm(x, W, b):
  return pl.pallas_call(
      _root_body,
      grid=(_NB,),
      in_specs=[_row_spec, _w_spec, _b_spec],
      out_specs=_row_spec,
      out_shape=jax.ShapeDtypeStruct((_N, _D), _F32),
  )(x, W, b.reshape(1, _D))


def kernel(x, edge_index, batch, W1l, b1l, W1r, W2l, b2l, W2r):
  edge_flat = edge_index.reshape(2 * _E)
  agg1, cnt = _sc_layer1(x, edge_flat)
  xr = _root_mm(x, W1r, b1l)
  h = pl.pallas_call(
      _tc1_body,
      grid=(_NB,),
      in_specs=[_agg_spec, _cnt_spec, _row_spec, _w_spec],
      out_specs=_row_spec,
      out_shape=jax.ShapeDtypeStruct((_N, _D), _F32),
  )(agg1, cnt.reshape(_NW, _NB, 1, _R), xr, W1l)
  (agg2,) = _sc_layer2(h, edge_flat)
  hr = _root_mm(h, W2r, b2l)
  pooled, h2 = pl.pallas_call(
      _tc2_body,
      grid=(_NB,),
      in_specs=[_agg_spec, _cnt_spec, _row_spec, _w_spec,
                pl.BlockSpec((1, 1, _R), lambda i: (i, 0, 0))],
      out_specs=(pl.BlockSpec((_G, _D), lambda i: (0, 0)), _row_spec),
      out_shape=(jax.ShapeDtypeStruct((_G, _D), _F32),
                 jax.ShapeDtypeStruct((_N, _D), _F32)),
      scratch_shapes=[pltpu.VMEM((_G, 1), _F32)],
  )(agg2, cnt.reshape(_NW, _NB, 1, _R), hr, W2l, batch.reshape(_NB, 1, _R))
  return (pooled, h2)
